# baseline (speedup 1.0000x reference)
import functools

import jax
import jax.numpy as jnp
from jax import lax
from jax.experimental import pallas as pl
from jax.experimental.pallas import tpu as pltpu
from jax.experimental.pallas import tpu_sc as plsc


_BS = 1024


def _tc_add_kernel(x_ref, pos_ref, o_ref):
    o_ref[...] = x_ref[...] + pos_ref[...][None, :, :]


def _tc_kernel(x, pos_table):
    batch, seq, d = x.shape
    bs = _BS if seq % _BS == 0 else seq
    grid = (seq // bs,)
    return pl.pallas_call(
        _tc_add_kernel,
        grid=grid,
        in_specs=[
            pl.BlockSpec((batch, bs, d), lambda i: (0, i, 0)),
            pl.BlockSpec((bs, d), lambda i: (i, 0)),
        ],
        out_specs=pl.BlockSpec((batch, bs, d), lambda i: (0, i, 0)),
        out_shape=jax.ShapeDtypeStruct((batch, seq, d), x.dtype),
    )(x, pos_table)


_NW = 32
_CH_ROWS = 32


def _sc_kernel(x, pos_table):
    batch, seq, d = x.shape
    nrows = batch * seq
    rows_per_w = nrows // _NW
    nchunks = rows_per_w // _CH_ROWS
    chunk = _CH_ROWS * d

    mesh = plsc.VectorSubcoreMesh(core_axis_name="c", subcore_axis_name="s")

    @functools.partial(
        pl.kernel,
        mesh=mesh,
        out_type=jax.ShapeDtypeStruct((nrows * d,), jnp.float32),
        scratch_types=[
            pltpu.VMEM((chunk,), jnp.float32),
            pltpu.VMEM((chunk,), jnp.float32),
            pltpu.VMEM((chunk,), jnp.float32),
            pltpu.VMEM((chunk,), jnp.float32),
            pltpu.SemaphoreType.DMA,
            pltpu.SemaphoreType.DMA,
            pltpu.SemaphoreType.DMA,
            pltpu.SemaphoreType.DMA,
        ],
    )
    def k(x_hbm, pos_hbm, o_hbm, xb0, xb1, pb0, pb1, si0, si1, so0, so1):
        xbufs, pbufs = (xb0, xb1), (pb0, pb1)
        sins, souts = (si0, si1), (so0, so1)
        wid = lax.axis_index("s") * 2 + lax.axis_index("c")
        base = wid * rows_per_w
        pbase = lax.rem(base, seq)

        def in_copies(g, b):
            r0 = (base + g * _CH_ROWS) * d
            p0 = (pbase + g * _CH_ROWS) * d
            cx = pltpu.make_async_copy(
                x_hbm.at[pl.ds(r0, chunk)], xbufs[b], sins[b])
            cp = pltpu.make_async_copy(
                pos_hbm.at[pl.ds(p0, chunk)], pbufs[b], sins[b])
            return cx, cp

        def out_copy(g, b):
            r0 = (base + g * _CH_ROWS) * d
            return pltpu.make_async_copy(
                xbufs[b], o_hbm.at[pl.ds(r0, chunk)], souts[b])

        cx, cp = in_copies(0, 0)
        cx.start()
        cp.start()
        for g in range(nchunks):
            b = g % 2
            cx, cp = in_copies(g, b)
            cx.wait()
            cp.wait()
            if g + 1 < nchunks:
                if g >= 1:
                    out_copy(g - 1, 1 - b).wait()
                ncx, ncp = in_copies(g + 1, 1 - b)
                ncx.start()
                ncp.start()

            xbuf, pbuf = xbufs[b], pbufs[b]

            @plsc.parallel_loop(0, chunk, step=16, unroll=8)
            def add(o):
                xbuf[pl.ds(o, 16)] = xbuf[pl.ds(o, 16)] + pbuf[pl.ds(o, 16)]

            out_copy(g, b).start()
        out_copy(nchunks - 2, nchunks % 2).wait()
        out_copy(nchunks - 1, 1 - nchunks % 2).wait()

    out = k(x.reshape(-1), pos_table.reshape(-1))
    return out.reshape(batch, seq, d)


def kernel(x, pos_table):
    return _sc_kernel(x, pos_table)


# speedup vs baseline: 4.8765x; 4.8765x over previous
import functools

import jax
import jax.numpy as jnp
from jax import lax
from jax.experimental import pallas as pl
from jax.experimental.pallas import tpu as pltpu
from jax.experimental.pallas import tpu_sc as plsc


_BS = 256


def _tc_add_kernel(x_ref, pos_ref, o_ref):
    o_ref[...] = x_ref[...] + pos_ref[...][None, :, :]


def _tc_kernel(x, pos_table):
    batch, seq, d = x.shape
    bs = _BS if seq % _BS == 0 else seq
    grid = (seq // bs,)
    return pl.pallas_call(
        _tc_add_kernel,
        grid=grid,
        in_specs=[
            pl.BlockSpec((batch, bs, d), lambda i: (0, i, 0)),
            pl.BlockSpec((bs, d), lambda i: (i, 0)),
        ],
        out_specs=pl.BlockSpec((batch, bs, d), lambda i: (0, i, 0)),
        out_shape=jax.ShapeDtypeStruct((batch, seq, d), x.dtype),
    )(x, pos_table)


_NW = 32
_CH_ROWS = 32


def _sc_kernel(x, pos_table):
    batch, seq, d = x.shape
    nrows = batch * seq
    rows_per_w = nrows // _NW
    nchunks = rows_per_w // _CH_ROWS
    chunk = _CH_ROWS * d

    mesh = plsc.VectorSubcoreMesh(core_axis_name="c", subcore_axis_name="s")

    @functools.partial(
        pl.kernel,
        mesh=mesh,
        out_type=jax.ShapeDtypeStruct((nrows * d,), jnp.float32),
        scratch_types=[
            pltpu.VMEM((chunk,), jnp.float32),
            pltpu.VMEM((chunk,), jnp.float32),
            pltpu.VMEM((chunk,), jnp.float32),
            pltpu.VMEM((chunk,), jnp.float32),
            pltpu.SemaphoreType.DMA,
            pltpu.SemaphoreType.DMA,
            pltpu.SemaphoreType.DMA,
            pltpu.SemaphoreType.DMA,
        ],
    )
    def k(x_hbm, pos_hbm, o_hbm, xb0, xb1, pb0, pb1, si0, si1, so0, so1):
        xbufs, pbufs = (xb0, xb1), (pb0, pb1)
        sins, souts = (si0, si1), (so0, so1)
        wid = lax.axis_index("s") * 2 + lax.axis_index("c")
        base = wid * rows_per_w
        pbase = lax.rem(base, seq)

        def in_copies(g, b):
            r0 = (base + g * _CH_ROWS) * d
            p0 = (pbase + g * _CH_ROWS) * d
            cx = pltpu.make_async_copy(
                x_hbm.at[pl.ds(r0, chunk)], xbufs[b], sins[b])
            cp = pltpu.make_async_copy(
                pos_hbm.at[pl.ds(p0, chunk)], pbufs[b], sins[b])
            return cx, cp

        def out_copy(g, b):
            r0 = (base + g * _CH_ROWS) * d
            return pltpu.make_async_copy(
                xbufs[b], o_hbm.at[pl.ds(r0, chunk)], souts[b])

        cx, cp = in_copies(0, 0)
        cx.start()
        cp.start()
        for g in range(nchunks):
            b = g % 2
            cx, cp = in_copies(g, b)
            cx.wait()
            cp.wait()
            if g + 1 < nchunks:
                if g >= 1:
                    out_copy(g - 1, 1 - b).wait()
                ncx, ncp = in_copies(g + 1, 1 - b)
                ncx.start()
                ncp.start()

            xbuf, pbuf = xbufs[b], pbufs[b]

            @plsc.parallel_loop(0, chunk, step=16, unroll=8)
            def add(o):
                xbuf[pl.ds(o, 16)] = xbuf[pl.ds(o, 16)] + pbuf[pl.ds(o, 16)]

            out_copy(g, b).start()
        out_copy(nchunks - 2, nchunks % 2).wait()
        out_copy(nchunks - 1, 1 - nchunks % 2).wait()

    out = k(x.reshape(-1), pos_table.reshape(-1))
    return out.reshape(batch, seq, d)


def kernel(x, pos_table):
    return _tc_kernel(x, pos_table)
